# baseline (speedup 1.0000x reference)
import functools

import jax
import jax.numpy as jnp
from jax import lax
from jax.experimental import pallas as pl
from jax.experimental.pallas import tpu as pltpu
from jax.experimental.pallas import tpu_sc as plsc

N = 10000
NROWS = 10240
DH = 64
DW = 128
NC, NS = 2, 16
NW = NC * NS
K = 128
RPT = NROWS // NS
C1 = 82
E1 = NW * K * C1
C2 = 80
NB = 4
C2B = C2 // NB
E2 = NW * K * C2
E2B = E2 // NB
R = 1024
G = NROWS // R
GEB = E2B // R


@functools.cache
def _sc_degree_kernel():
    mesh = plsc.VectorSubcoreMesh(core_axis_name="c", subcore_axis_name="s")
    return functools.partial(
        pl.kernel, mesh=mesh,
        out_type=jax.ShapeDtypeStruct((NC * NROWS, DW), jnp.float32),
        scratch_types=[
            pltpu.VMEM((C1, K), jnp.int32),
            pltpu.VMEM((K, DW), jnp.float32),
            pltpu.VMEM_SHARED((NROWS, DW), jnp.float32),
        ],
    )(_sc_degree_body)


def _sc_degree(dst1):
    return _sc_degree_kernel()(dst1)


def _sc_degree_body(dst_hbm, out_hbm, didx_b, ones_v, deg_sh):
    c = lax.axis_index("c")
    s = lax.axis_index("s")
    wid = c * NS + s

    def zrow(i, carry):
        for j in range(DW // 16):
            ones_v[i, pl.ds(j * 16, 16)] = jnp.zeros((16,), jnp.float32)
        return carry
    lax.fori_loop(0, K, zrow, 0)
    for k in range(RPT // K):
        pltpu.sync_copy(ones_v, deg_sh.at[pl.ds(s * RPT + k * K, K)])

    def orow(i, carry):
        for j in range(DW // 16):
            ones_v[i, pl.ds(j * 16, 16)] = jnp.ones((16,), jnp.float32)
        return carry
    lax.fori_loop(0, K, orow, 0)
    plsc.subcore_barrier()

    pltpu.sync_copy(dst_hbm.at[wid], didx_b)

    def chunk(i, carry):
        pltpu.sync_copy(ones_v, deg_sh.at[didx_b.at[i]], add=True)
        return carry
    lax.fori_loop(0, C1, chunk, 0)
    plsc.subcore_barrier()
    pltpu.sync_copy(deg_sh.at[pl.ds(s * RPT, RPT)],
                    out_hbm.at[pl.ds(c * NROWS + s * RPT, RPT)])


@functools.cache
def _sc_agg_kernel():
    mesh = plsc.VectorSubcoreMesh(core_axis_name="c", subcore_axis_name="s")
    return functools.partial(
        pl.kernel, mesh=mesh,
        out_type=jax.ShapeDtypeStruct((NC * NROWS, DW), jnp.float32),
        scratch_types=[
            pltpu.VMEM((K,), jnp.int32),
            pltpu.VMEM((K,), jnp.int32),
            pltpu.VMEM((K,), jnp.int32),
            pltpu.VMEM((K,), jnp.int32),
            pltpu.VMEM((K, DW), jnp.float32),
            pltpu.VMEM((K, DW), jnp.float32),
            pltpu.VMEM_SHARED((NROWS, DW), jnp.float32),
            pltpu.SemaphoreType.DMA,
            pltpu.SemaphoreType.DMA,
        ],
    )(_sc_agg_body)


def _sc_agg(hw, src1, dst1):
    return _sc_agg_kernel()(hw, src1, dst1)


def _sc_agg_body(hw_hbm, src_hbm, dst_hbm, out_hbm, si0, si1, di0, di1,
                 rows0, rows1, agg_sh, g0, g1):
    c = lax.axis_index("c")
    s = lax.axis_index("s")
    wid = c * NS + s
    rows = (rows0, rows1)
    gsem = (g0, g1)
    sibuf = (si0, si1)
    dibuf = (di0, di1)

    def zrow(i, carry):
        for j in range(DW // 16):
            rows0[i, pl.ds(j * 16, 16)] = jnp.zeros((16,), jnp.float32)
        return carry
    lax.fori_loop(0, K, zrow, 0)
    for k in range(RPT // K):
        pltpu.sync_copy(rows0, agg_sh.at[pl.ds(s * RPT + k * K, K)])
    plsc.subcore_barrier()

    rows = (rows0, rows1)
    gsem = (g0, g1)
    sib = (si0, si1)
    dib = (di0, di1)

    def load_and_gather(j, b):
        base = wid * (C1 * K) + j * K
        pltpu.sync_copy(src_hbm.at[pl.ds(base, K)], sib[b])
        pltpu.sync_copy(dst_hbm.at[pl.ds(base, K)], dib[b])
        pltpu.async_copy(hw_hbm.at[sib[b]], rows[b], gsem[b])

    def wait_g(b):
        pltpu.make_async_copy(hw_hbm.at[sib[b]], rows[b], gsem[b]).wait()

    def scatter(b):
        pltpu.sync_copy(rows[b], agg_sh.at[dib[b]], add=True)

    load_and_gather(0, 0)

    def pair(p, carry):
        j0 = 2 * p
        wait_g(0)
        load_and_gather(j0 + 1, 1)
        scatter(0)
        wait_g(1)

        @pl.when(j0 + 2 < C1)
        def _next():
            load_and_gather(j0 + 2, 0)
        scatter(1)
        return carry
    lax.fori_loop(0, C1 // 2, pair, 0)
    plsc.subcore_barrier()
    pltpu.sync_copy(agg_sh.at[pl.ds(s * RPT, RPT)],
                    out_hbm.at[pl.ds(c * NROWS + s * RPT, RPT)])


@functools.cache
def _sc_edge_gather_kernel():
    mesh = plsc.VectorSubcoreMesh(core_axis_name="c", subcore_axis_name="s")
    return functools.partial(
        pl.kernel, mesh=mesh,
        out_type=(jax.ShapeDtypeStruct((E2B, DW), jnp.float32),
                  jax.ShapeDtypeStruct((E2B, DW), jnp.float32)),
        scratch_types=[
            pltpu.VMEM((K,), jnp.int32),
            pltpu.VMEM((K,), jnp.int32),
            pltpu.VMEM((K,), jnp.int32),
            pltpu.VMEM((K,), jnp.int32),
            pltpu.VMEM((K, DW), jnp.float32),
            pltpu.VMEM((K, DW), jnp.float32),
            pltpu.VMEM((K, DW), jnp.float32),
            pltpu.VMEM((K, DW), jnp.float32),
            pltpu.SemaphoreType.DMA,
            pltpu.SemaphoreType.DMA,
            pltpu.SemaphoreType.DMA,
            pltpu.SemaphoreType.DMA,
        ],
    )(_sc_edge_gather_body)


def _sc_edge_gather(h3, sidx, tidx):
    return _sc_edge_gather_kernel()(h3, sidx, tidx)


def _sc_edge_gather_body(h_hbm, sidx_hbm, tidx_hbm, eea_hbm, eeb_hbm,
                         si0, si1, ti0, ti1, ra0, ra1, rb0, rb1,
                         ga0, ga1, gb0, gb1):
    c = lax.axis_index("c")
    s = lax.axis_index("s")
    wid = c * NS + s
    sib = (si0, si1)
    tib = (ti0, ti1)
    ra = (ra0, ra1)
    rb = (rb0, rb1)
    gsa = (ga0, ga1)
    gsb = (gb0, gb1)

    def load_and_gather(j, b):
        base = wid * (C2B * K) + j * K
        pltpu.sync_copy(sidx_hbm.at[pl.ds(base, K)], sib[b])
        pltpu.sync_copy(tidx_hbm.at[pl.ds(base, K)], tib[b])
        pltpu.async_copy(h_hbm.at[sib[b]], ra[b], gsa[b])
        pltpu.async_copy(h_hbm.at[tib[b]], rb[b], gsb[b])

    def wait_g(b):
        pltpu.make_async_copy(h_hbm.at[sib[b]], ra[b], gsa[b]).wait()
        pltpu.make_async_copy(h_hbm.at[tib[b]], rb[b], gsb[b]).wait()

    def write(j, b):
        base = wid * (C2B * K) + j * K
        pltpu.sync_copy(ra[b], eea_hbm.at[pl.ds(base, K)])
        pltpu.sync_copy(rb[b], eeb_hbm.at[pl.ds(base, K)])

    load_and_gather(0, 0)

    def pair(p, carry):
        j0 = 2 * p
        wait_g(0)
        load_and_gather(j0 + 1, 1)
        write(j0, 0)
        wait_g(1)

        @pl.when(j0 + 2 < C2B)
        def _next():
            load_and_gather(j0 + 2, 0)
        write(j0 + 1, 1)
        return carry
    lax.fori_loop(0, C2B // 2, pair, 0)


def _dinv_block(da, db):
    deg = da[:, 0:1] + db[:, 0:1]
    return jnp.where(deg > 0, lax.rsqrt(deg), 0.0)


def _ln(v, g, b):
    m = jnp.mean(v, axis=-1, keepdims=True)
    var = jnp.mean((v - m) ** 2, axis=-1, keepdims=True)
    return (v - m) * lax.rsqrt(var + 1e-5) * g + b


def _dot(a, b):
    return jnp.dot(a, b, preferred_element_type=jnp.float32)


def _full(shape):
    return pl.BlockSpec(shape, lambda i: (0,) * len(shape))


def _tc_pre(x_pad, w0, p0):
    def body(x_ref, w_ref, p_ref, xw_ref, id_ref):
        xb = x_ref[...]
        xw_ref[...] = _dot(xb, w_ref[...])
        id_ref[...] = _dot(xb, p_ref[...])

    return pl.pallas_call(
        body,
        grid=(G,),
        in_specs=[pl.BlockSpec((R, 128), lambda i: (i, 0)),
                  _full((128, DH)), _full((128, DH))],
        out_specs=[pl.BlockSpec((R, DH), lambda i: (i, 0)),
                   pl.BlockSpec((R, DH), lambda i: (i, 0))],
        out_shape=[jax.ShapeDtypeStruct((NROWS, DH), jnp.float32),
                   jax.ShapeDtypeStruct((NROWS, DH), jnp.float32)],
    )(x_pad, w0, p0)


def _tc_scale(xw, degs):
    def body(xw_ref, da, db, hw_ref):
        dinv = _dinv_block(da, db)
        hw_ref[...] = jnp.concatenate(
            [xw_ref[...] * dinv, jnp.zeros((R, DW - DH), jnp.float32)],
            axis=1)

    return pl.pallas_call(
        body,
        grid=(G,),
        in_specs=[pl.BlockSpec((R, DH), lambda i: (i, 0)),
                  pl.BlockSpec((R, DW), lambda i: (i, 0)),
                  pl.BlockSpec((R, DW), lambda i: (i + G, 0))],
        out_specs=[pl.BlockSpec((R, DW), lambda i: (i, 0))],
        out_shape=[jax.ShapeDtypeStruct((NROWS, DW), jnp.float32)],
    )(xw, degs, degs)[0]


def _tc_post(aggs, degs, ident, b, g, be, wnext, hw_prev):
    def body(aa, ab, da, db, idn, b_r, g_r, be_r, w_r, hp_r, h_ref, hw_ref):
        dinv = _dinv_block(da, db)
        agg = (aa[:, :DH] + ab[:, :DH]) * dinv + b_r[...]
        h = jnp.maximum(_ln(agg, g_r[...], be_r[...]) + idn[...], 0.0)
        h_ref[...] = h
        hw = _dot(h, w_r[...]) * dinv
        hw_ref[...] = jnp.concatenate(
            [hw, jnp.zeros((R, DW - DH), jnp.float32)], axis=1)

    return pl.pallas_call(
        body,
        grid=(G,),
        in_specs=[pl.BlockSpec((R, DW), lambda i: (i, 0)),
                  pl.BlockSpec((R, DW), lambda i: (i + G, 0)),
                  pl.BlockSpec((R, DW), lambda i: (i, 0)),
                  pl.BlockSpec((R, DW), lambda i: (i + G, 0)),
                  pl.BlockSpec((R, DH), lambda i: (i, 0)),
                  _full((1, DH)), _full((1, DH)), _full((1, DH)),
                  _full((DH, DH)),
                  pl.BlockSpec((R, DW), lambda i: (i, 0))],
        out_specs=[pl.BlockSpec((R, DH), lambda i: (i, 0)),
                   pl.BlockSpec((R, DW), lambda i: (i, 0))],
        out_shape=[jax.ShapeDtypeStruct((NROWS, DH), jnp.float32),
                   jax.ShapeDtypeStruct((NROWS, DW), jnp.float32)],
        input_output_aliases={9: 1},
    )(aggs, aggs, degs, degs, ident, b, g, be, wnext, hw_prev)


def _tc_post_h3(aggs, degs, ident, b, g, be, hw_prev):
    def body(aa, ab, da, db, idn, b_r, g_r, be_r, hp_r, h_ref):
        dinv = _dinv_block(da, db)
        agg = (aa[:, :DH] + ab[:, :DH]) * dinv + b_r[...]
        h = jnp.maximum(_ln(agg, g_r[...], be_r[...]) + idn[...], 0.0)
        h_ref[...] = jnp.concatenate(
            [h, jnp.zeros((R, DW - DH), jnp.float32)], axis=1)

    return pl.pallas_call(
        body,
        grid=(G,),
        in_specs=[pl.BlockSpec((R, DW), lambda i: (i, 0)),
                  pl.BlockSpec((R, DW), lambda i: (i + G, 0)),
                  pl.BlockSpec((R, DW), lambda i: (i, 0)),
                  pl.BlockSpec((R, DW), lambda i: (i + G, 0)),
                  pl.BlockSpec((R, DH), lambda i: (i, 0)),
                  _full((1, DH)), _full((1, DH)), _full((1, DH)),
                  pl.BlockSpec((R, DW), lambda i: (i, 0))],
        out_specs=[pl.BlockSpec((R, DW), lambda i: (i, 0))],
        out_shape=[jax.ShapeDtypeStruct((NROWS, DW), jnp.float32)],
        input_output_aliases={8: 0},
    )(aggs, aggs, degs, degs, ident, b, g, be, hw_prev)[0]


def _tc_place_head(h3, place_p):
    (w1, b1, g1, be1), (w2, b2, g2, be2), (w3, b3) = place_p

    def body(hr, w1r, b1r, g1r, be1r, w2r, b2r, g2r, be2r, w3r, b3r,
             pl_ref):
        h = hr[:, :DH]
        t = jnp.maximum(_ln(_dot(h, w1r[...]) + b1r[...], g1r[...],
                            be1r[...]), 0.0)
        t = jnp.maximum(_ln(_dot(t, w2r[...]) + b2r[...], g2r[...],
                            be2r[...]), 0.0)
        pl_ref[...] = jnp.clip(_dot(t, w3r[...]) + b3r[...], -20.0, 20.0)

    return pl.pallas_call(
        body,
        grid=(G,),
        in_specs=[pl.BlockSpec((R, DW), lambda i: (i, 0)),
                  _full((DH, 64)), _full((1, 64)), _full((1, 64)),
                  _full((1, 64)),
                  _full((64, 32)), _full((1, 32)), _full((1, 32)),
                  _full((1, 32)),
                  _full((32, 8)), _full((1, 8))],
        out_specs=[pl.BlockSpec((R, 8), lambda i: (i, 0))],
        out_shape=[jax.ShapeDtypeStruct((NROWS, 8), jnp.float32)],
    )(h3, w1, b1, g1, be1, w2, b2, g2, be2, w3, b3)[0]


def _tc_edge_mlp(eea, eeb, edge_p, army_p):
    (we1a, we1b, be1, ge1, bee1), (we2, be2, ge2, bee2), (we3, be3) = edge_p
    (wa1a, wa1b, ba1, ga1, baa1), (wa2, ba2, ga2, baa2), (wa3, ba3) = army_p

    def body(ea, eb,
             e1a, e1b, e1bias, e1g, e1be, e2w, e2b, e2g, e2be, e3w, e3b,
             a1a, a1b, a1bias, a1g, a1be, a2w, a2b, a2g, a2be, a3w, a3b,
             att_ref, army_ref):
        bf = jnp.bfloat16
        a = jnp.clip(ea[:, :DH], -10.0, 10.0).astype(bf)
        b = jnp.clip(eb[:, :DH], -10.0, 10.0).astype(bf)
        t = _dot(a, e1a[...]) + _dot(b, e1b[...]) + e1bias[...]
        t = jnp.maximum(_ln(t, e1g[...], e1be[...]), 0.0)
        t = jnp.maximum(_ln(_dot(t.astype(bf), e2w[...]) + e2b[...],
                            e2g[...], e2be[...]), 0.0)
        att = jnp.clip(_dot(t.astype(bf), e3w[...]) + e3b[...], -20.0, 20.0)
        att_ref[...] = att[:, 0]
        u = _dot(a, a1a[...]) + _dot(b, a1b[...]) + a1bias[...]
        u = jnp.maximum(_ln(u, a1g[...], a1be[...]), 0.0)
        u = jnp.maximum(_ln(_dot(u.astype(bf), a2w[...]) + a2b[...],
                            a2g[...], a2be[...]), 0.0)
        army = jnp.clip(_dot(u.astype(bf), a3w[...]) + a3b[...], -20.0,
                        20.0)
        army_ref[...] = army[:, :4]

    return pl.pallas_call(
        body,
        grid=(GEB,),
        in_specs=[pl.BlockSpec((R, DW), lambda i: (i, 0)),
                  pl.BlockSpec((R, DW), lambda i: (i, 0)),
                  _full((DH, 64)), _full((DH, 64)), _full((1, 64)),
                  _full((1, 64)), _full((1, 64)),
                  _full((64, 32)), _full((1, 32)), _full((1, 32)),
                  _full((1, 32)),
                  _full((32, 8)), _full((1, 8)),
                  _full((DH, 128)), _full((DH, 128)), _full((1, 128)),
                  _full((1, 128)), _full((1, 128)),
                  _full((128, 64)), _full((1, 64)), _full((1, 64)),
                  _full((1, 64)),
                  _full((64, 8)), _full((1, 8))],
        out_specs=[pl.BlockSpec((R,), lambda i: (i,)),
                   pl.BlockSpec((R, 4), lambda i: (i, 0))],
        out_shape=[jax.ShapeDtypeStruct((E2B,), jnp.float32),
                   jax.ShapeDtypeStruct((E2B, 4), jnp.float32)],
    )(eea, eeb,
      we1a, we1b, be1, ge1, bee1, we2, be2, ge2, bee2, we3, be3,
      wa1a, wa1b, ba1, ga1, baa1, wa2, ba2, ga2, baa2, wa3, ba3)


def _row(v):
    return v.reshape(1, -1)


def _pad_cols(w, b, cols):
    wp = jnp.zeros((w.shape[0], cols), w.dtype).at[:, :w.shape[1]].set(w)
    bp = jnp.zeros((cols,), b.dtype).at[:b.shape[0]].set(b)
    return wp, _row(bp)


def _head3(p, cols):
    l1, l2, l3 = p
    w3, b3 = _pad_cols(l3["W"], l3["b"], cols)
    return ((l1["W"], _row(l1["b"]), _row(l1["g"]), _row(l1["be"])),
            (l2["W"], _row(l2["b"]), _row(l2["g"]), _row(l2["be"])),
            (w3, b3))


def _head3_split(p, cols):
    l1, l2, l3 = p
    bf = jnp.bfloat16
    w1 = l1["W"].astype(bf)
    w3, b3 = _pad_cols(l3["W"], l3["b"], cols)
    return ((w1[:DH], w1[DH:], _row(l1["b"]), _row(l1["g"]), _row(l1["be"])),
            (l2["W"].astype(bf), _row(l2["b"]), _row(l2["g"]),
             _row(l2["be"])),
            (w3.astype(bf), b3))


def kernel(x, params, action_edges, edge_index):
    n = x.shape[0]
    ne = action_edges.shape[0]
    ei = edge_index.astype(jnp.int32)
    loop = jnp.arange(n, dtype=jnp.int32)
    npad1 = E1 - (ei.shape[1] + n)
    spread = N + jax.lax.rem(jnp.arange(npad1, dtype=jnp.int32),
                             jnp.int32(NROWS - N))
    src1 = jnp.concatenate([ei[0], loop, spread])
    dst1 = jnp.concatenate([ei[1], loop, spread])
    dst3 = dst1.reshape(NW, C1, K)
    ae = action_edges.astype(jnp.int32)
    npad2 = E2 - ne
    spread2 = jax.lax.rem(jnp.arange(npad2, dtype=jnp.int32), jnp.int32(n))
    sidx = jnp.concatenate([jnp.clip(ae[:, 0], 0, n - 1), spread2])
    tidx = jnp.concatenate([jnp.clip(ae[:, 1], 0, n - 1), spread2])
    x_pad = jnp.zeros((NROWS, x.shape[1]), x.dtype).at[:n].set(x)

    gcn = params["gcn"]
    degs = _sc_degree(dst3)
    xw0, id0 = _tc_pre(x_pad, gcn[0]["W"], gcn[0]["P"])
    hw0 = _tc_scale(xw0, degs)
    agg0 = _sc_agg(hw0, src1, dst1)
    h1, hw1 = _tc_post(agg0, degs, id0, _row(gcn[0]["b"]), _row(gcn[0]["g"]),
                       _row(gcn[0]["be"]), gcn[1]["W"], hw0)
    agg1 = _sc_agg(hw1, src1, dst1)
    h2, hw2 = _tc_post(agg1, degs, h1, _row(gcn[1]["b"]), _row(gcn[1]["g"]),
                       _row(gcn[1]["be"]), gcn[2]["W"], hw1)
    agg2 = _sc_agg(hw2, src1, dst1)
    h3 = _tc_post_h3(agg2, degs, h2, _row(gcn[2]["b"]), _row(gcn[2]["g"]),
                     _row(gcn[2]["be"]), hw2)
    edge_p = _head3_split(params["edge"], 8)
    army_p = _head3_split(params["army"], 8)
    ees = [_sc_edge_gather(h3, sidx[k * E2B:(k + 1) * E2B],
                           tidx[k * E2B:(k + 1) * E2B]) for k in range(NB)]
    place8 = _tc_place_head(h3, _head3(params["place"], 8))
    outs = [_tc_edge_mlp(ea, eb, edge_p, army_p) for ea, eb in ees]

    placement_logits = place8[:n, 0]
    attack_logits = jnp.concatenate([o[0] for o in outs])[:ne]
    army_logits = jnp.concatenate([o[1] for o in outs])[:ne]
    return (placement_logits, attack_logits, army_logits)


# speedup vs baseline: 1.0517x; 1.0517x over previous
import functools

import jax
import jax.numpy as jnp
from jax import lax
from jax.experimental import pallas as pl
from jax.experimental.pallas import tpu as pltpu
from jax.experimental.pallas import tpu_sc as plsc

N = 10000
NROWS = 10240
DH = 64
DW = 128
NC, NS = 2, 16
NW = NC * NS
K = 128
RPT = NROWS // NS
C1 = 82
E1 = NW * K * C1
C2 = 80
NB = 4
C2B = C2 // NB
E2 = NW * K * C2
E2B = E2 // NB
R = 1024
G = NROWS // R
GEB = E2B // R


@functools.cache
def _sc_degree_kernel():
    mesh = plsc.VectorSubcoreMesh(core_axis_name="c", subcore_axis_name="s")
    return functools.partial(
        pl.kernel, mesh=mesh,
        out_type=jax.ShapeDtypeStruct((NC * NROWS, DW), jnp.float32),
        scratch_types=[
            pltpu.VMEM((C1, K), jnp.int32),
            pltpu.VMEM((K, DW), jnp.float32),
            pltpu.VMEM_SHARED((NROWS, DW), jnp.float32),
        ],
    )(_sc_degree_body)


def _sc_degree(dst1):
    return _sc_degree_kernel()(dst1)


def _sc_degree_body(dst_hbm, out_hbm, didx_b, ones_v, deg_sh):
    c = lax.axis_index("c")
    s = lax.axis_index("s")
    wid = c * NS + s

    def zrow(i, carry):
        for j in range(DW // 16):
            ones_v[i, pl.ds(j * 16, 16)] = jnp.zeros((16,), jnp.float32)
        return carry
    lax.fori_loop(0, K, zrow, 0)
    for k in range(RPT // K):
        pltpu.sync_copy(ones_v, deg_sh.at[pl.ds(s * RPT + k * K, K)])

    def orow(i, carry):
        for j in range(DW // 16):
            ones_v[i, pl.ds(j * 16, 16)] = jnp.ones((16,), jnp.float32)
        return carry
    lax.fori_loop(0, K, orow, 0)
    plsc.subcore_barrier()

    pltpu.sync_copy(dst_hbm.at[wid], didx_b)

    def chunk(i, carry):
        pltpu.sync_copy(ones_v, deg_sh.at[didx_b.at[i]], add=True)
        return carry
    lax.fori_loop(0, C1, chunk, 0)
    plsc.subcore_barrier()
    pltpu.sync_copy(deg_sh.at[pl.ds(s * RPT, RPT)],
                    out_hbm.at[pl.ds(c * NROWS + s * RPT, RPT)])


@functools.cache
def _sc_agg_kernel():
    mesh = plsc.VectorSubcoreMesh(core_axis_name="c", subcore_axis_name="s")
    return functools.partial(
        pl.kernel, mesh=mesh,
        out_type=jax.ShapeDtypeStruct((NC * NROWS, DW), jnp.float32),
        scratch_types=[
            pltpu.VMEM((K,), jnp.int32),
            pltpu.VMEM((K,), jnp.int32),
            pltpu.VMEM((K,), jnp.int32),
            pltpu.VMEM((K,), jnp.int32),
            pltpu.VMEM((K, DW), jnp.float32),
            pltpu.VMEM((K, DW), jnp.float32),
            pltpu.VMEM_SHARED((NROWS, DW), jnp.float32),
            pltpu.SemaphoreType.DMA,
            pltpu.SemaphoreType.DMA,
        ],
    )(_sc_agg_body)


def _sc_agg(hw, src1, dst1):
    return _sc_agg_kernel()(hw, src1, dst1)


def _sc_agg_body(hw_hbm, src_hbm, dst_hbm, out_hbm, si0, si1, di0, di1,
                 rows0, rows1, agg_sh, g0, g1):
    c = lax.axis_index("c")
    s = lax.axis_index("s")
    wid = c * NS + s
    rows = (rows0, rows1)
    gsem = (g0, g1)
    sibuf = (si0, si1)
    dibuf = (di0, di1)

    def zrow(i, carry):
        for j in range(DW // 16):
            rows0[i, pl.ds(j * 16, 16)] = jnp.zeros((16,), jnp.float32)
        return carry
    lax.fori_loop(0, K, zrow, 0)
    for k in range(RPT // K):
        pltpu.sync_copy(rows0, agg_sh.at[pl.ds(s * RPT + k * K, K)])
    plsc.subcore_barrier()

    rows = (rows0, rows1)
    gsem = (g0, g1)
    sib = (si0, si1)
    dib = (di0, di1)

    def load_and_gather(j, b):
        base = wid * (C1 * K) + j * K
        pltpu.sync_copy(src_hbm.at[pl.ds(base, K)], sib[b])
        pltpu.sync_copy(dst_hbm.at[pl.ds(base, K)], dib[b])
        pltpu.async_copy(hw_hbm.at[sib[b]], rows[b], gsem[b])

    def wait_g(b):
        pltpu.make_async_copy(hw_hbm.at[sib[b]], rows[b], gsem[b]).wait()

    def scatter(b):
        pltpu.sync_copy(rows[b], agg_sh.at[dib[b]], add=True)

    load_and_gather(0, 0)

    def pair(p, carry):
        j0 = 2 * p
        wait_g(0)
        load_and_gather(j0 + 1, 1)
        scatter(0)
        wait_g(1)

        @pl.when(j0 + 2 < C1)
        def _next():
            load_and_gather(j0 + 2, 0)
        scatter(1)
        return carry
    lax.fori_loop(0, C1 // 2, pair, 0)
    plsc.subcore_barrier()
    pltpu.sync_copy(agg_sh.at[pl.ds(s * RPT, RPT)],
                    out_hbm.at[pl.ds(c * NROWS + s * RPT, RPT)])


@functools.cache
def _sc_edge_gather_kernel():
    mesh = plsc.VectorSubcoreMesh(core_axis_name="c", subcore_axis_name="s")
    return functools.partial(
        pl.kernel, mesh=mesh,
        out_type=(jax.ShapeDtypeStruct((E2B, DW), jnp.float32),
                  jax.ShapeDtypeStruct((E2B, DW), jnp.float32)),
        scratch_types=[
            pltpu.VMEM((K,), jnp.int32),
            pltpu.VMEM((K,), jnp.int32),
            pltpu.VMEM((K,), jnp.int32),
            pltpu.VMEM((K,), jnp.int32),
            pltpu.VMEM((K, DW), jnp.float32),
            pltpu.VMEM((K, DW), jnp.float32),
            pltpu.VMEM((K, DW), jnp.float32),
            pltpu.VMEM((K, DW), jnp.float32),
            pltpu.SemaphoreType.DMA,
            pltpu.SemaphoreType.DMA,
            pltpu.SemaphoreType.DMA,
            pltpu.SemaphoreType.DMA,
        ],
    )(_sc_edge_gather_body)


def _sc_edge_gather(h3, sidx, tidx):
    return _sc_edge_gather_kernel()(h3, sidx, tidx)


def _sc_edge_gather_body(h_hbm, sidx_hbm, tidx_hbm, eea_hbm, eeb_hbm,
                         si0, si1, ti0, ti1, ra0, ra1, rb0, rb1,
                         ga0, ga1, gb0, gb1):
    c = lax.axis_index("c")
    s = lax.axis_index("s")
    wid = c * NS + s
    sib = (si0, si1)
    tib = (ti0, ti1)
    ra = (ra0, ra1)
    rb = (rb0, rb1)
    gsa = (ga0, ga1)
    gsb = (gb0, gb1)

    def load_and_gather(j, b):
        base = wid * (C2B * K) + j * K
        pltpu.sync_copy(sidx_hbm.at[pl.ds(base, K)], sib[b])
        pltpu.sync_copy(tidx_hbm.at[pl.ds(base, K)], tib[b])
        pltpu.async_copy(h_hbm.at[sib[b]], ra[b], gsa[b])
        pltpu.async_copy(h_hbm.at[tib[b]], rb[b], gsb[b])

    def wait_g(b):
        pltpu.make_async_copy(h_hbm.at[sib[b]], ra[b], gsa[b]).wait()
        pltpu.make_async_copy(h_hbm.at[tib[b]], rb[b], gsb[b]).wait()

    def write(j, b):
        base = wid * (C2B * K) + j * K
        pltpu.sync_copy(ra[b], eea_hbm.at[pl.ds(base, K)])
        pltpu.sync_copy(rb[b], eeb_hbm.at[pl.ds(base, K)])

    load_and_gather(0, 0)

    def pair(p, carry):
        j0 = 2 * p
        wait_g(0)
        load_and_gather(j0 + 1, 1)
        write(j0, 0)
        wait_g(1)

        @pl.when(j0 + 2 < C2B)
        def _next():
            load_and_gather(j0 + 2, 0)
        write(j0 + 1, 1)
        return carry
    lax.fori_loop(0, C2B // 2, pair, 0)


def _dinv_block(da, db):
    deg = da[:, 0:1] + db[:, 0:1]
    return jnp.where(deg > 0, lax.rsqrt(deg), 0.0)


def _ln(v, g, b):
    m = jnp.mean(v, axis=-1, keepdims=True)
    var = jnp.mean((v - m) ** 2, axis=-1, keepdims=True)
    return (v - m) * lax.rsqrt(var + 1e-5) * g + b


def _dot(a, b):
    return jnp.dot(a, b, preferred_element_type=jnp.float32)


def _full(shape):
    return pl.BlockSpec(shape, lambda i: (0,) * len(shape))


def _tc_pre(x_pad, w0, p0):
    def body(x_ref, w_ref, p_ref, xw_ref, id_ref):
        xb = x_ref[...]
        xw_ref[...] = _dot(xb, w_ref[...])
        id_ref[...] = _dot(xb, p_ref[...])

    return pl.pallas_call(
        body,
        grid=(G,),
        in_specs=[pl.BlockSpec((R, 128), lambda i: (i, 0)),
                  _full((128, DH)), _full((128, DH))],
        out_specs=[pl.BlockSpec((R, DH), lambda i: (i, 0)),
                   pl.BlockSpec((R, DH), lambda i: (i, 0))],
        out_shape=[jax.ShapeDtypeStruct((NROWS, DH), jnp.float32),
                   jax.ShapeDtypeStruct((NROWS, DH), jnp.float32)],
    )(x_pad, w0, p0)


def _tc_scale(xw, degs):
    def body(xw_ref, da, db, hw_ref):
        dinv = _dinv_block(da, db)
        hw_ref[...] = jnp.concatenate(
            [xw_ref[...] * dinv, jnp.zeros((R, DW - DH), jnp.float32)],
            axis=1)

    return pl.pallas_call(
        body,
        grid=(G,),
        in_specs=[pl.BlockSpec((R, DH), lambda i: (i, 0)),
                  pl.BlockSpec((R, DW), lambda i: (i, 0)),
                  pl.BlockSpec((R, DW), lambda i: (i + G, 0))],
        out_specs=[pl.BlockSpec((R, DW), lambda i: (i, 0))],
        out_shape=[jax.ShapeDtypeStruct((NROWS, DW), jnp.float32)],
    )(xw, degs, degs)[0]


def _tc_post(aggs, degs, ident, b, g, be, wnext, hw_prev):
    def body(aa, ab, da, db, idn, b_r, g_r, be_r, w_r, hp_r, h_ref, hw_ref):
        dinv = _dinv_block(da, db)
        agg = (aa[:, :DH] + ab[:, :DH]) * dinv + b_r[...]
        h = jnp.maximum(_ln(agg, g_r[...], be_r[...]) + idn[...], 0.0)
        h_ref[...] = h
        hw = _dot(h, w_r[...]) * dinv
        hw_ref[...] = jnp.concatenate(
            [hw, jnp.zeros((R, DW - DH), jnp.float32)], axis=1)

    return pl.pallas_call(
        body,
        grid=(G,),
        in_specs=[pl.BlockSpec((R, DW), lambda i: (i, 0)),
                  pl.BlockSpec((R, DW), lambda i: (i + G, 0)),
                  pl.BlockSpec((R, DW), lambda i: (i, 0)),
                  pl.BlockSpec((R, DW), lambda i: (i + G, 0)),
                  pl.BlockSpec((R, DH), lambda i: (i, 0)),
                  _full((1, DH)), _full((1, DH)), _full((1, DH)),
                  _full((DH, DH)),
                  pl.BlockSpec((R, DW), lambda i: (i, 0))],
        out_specs=[pl.BlockSpec((R, DH), lambda i: (i, 0)),
                   pl.BlockSpec((R, DW), lambda i: (i, 0))],
        out_shape=[jax.ShapeDtypeStruct((NROWS, DH), jnp.float32),
                   jax.ShapeDtypeStruct((NROWS, DW), jnp.float32)],
        input_output_aliases={9: 1},
    )(aggs, aggs, degs, degs, ident, b, g, be, wnext, hw_prev)


def _tc_post_h3(aggs, degs, ident, b, g, be, hw_prev):
    def body(aa, ab, da, db, idn, b_r, g_r, be_r, hp_r, h_ref):
        dinv = _dinv_block(da, db)
        agg = (aa[:, :DH] + ab[:, :DH]) * dinv + b_r[...]
        h = jnp.maximum(_ln(agg, g_r[...], be_r[...]) + idn[...], 0.0)
        h_ref[...] = jnp.concatenate(
            [h, jnp.zeros((R, DW - DH), jnp.float32)], axis=1)

    return pl.pallas_call(
        body,
        grid=(G,),
        in_specs=[pl.BlockSpec((R, DW), lambda i: (i, 0)),
                  pl.BlockSpec((R, DW), lambda i: (i + G, 0)),
                  pl.BlockSpec((R, DW), lambda i: (i, 0)),
                  pl.BlockSpec((R, DW), lambda i: (i + G, 0)),
                  pl.BlockSpec((R, DH), lambda i: (i, 0)),
                  _full((1, DH)), _full((1, DH)), _full((1, DH)),
                  pl.BlockSpec((R, DW), lambda i: (i, 0))],
        out_specs=[pl.BlockSpec((R, DW), lambda i: (i, 0))],
        out_shape=[jax.ShapeDtypeStruct((NROWS, DW), jnp.float32)],
        input_output_aliases={8: 0},
    )(aggs, aggs, degs, degs, ident, b, g, be, hw_prev)[0]


def _tc_place_head(h3, place_p):
    (w1, b1, g1, be1), (w2, b2, g2, be2), (w3, b3) = place_p

    def body(hr, w1r, b1r, g1r, be1r, w2r, b2r, g2r, be2r, w3r, b3r,
             pl_ref):
        h = hr[:, :DH]
        t = jnp.maximum(_ln(_dot(h, w1r[...]) + b1r[...], g1r[...],
                            be1r[...]), 0.0)
        t = jnp.maximum(_ln(_dot(t, w2r[...]) + b2r[...], g2r[...],
                            be2r[...]), 0.0)
        pl_ref[...] = jnp.clip(_dot(t, w3r[...]) + b3r[...], -20.0, 20.0)

    return pl.pallas_call(
        body,
        grid=(G,),
        in_specs=[pl.BlockSpec((R, DW), lambda i: (i, 0)),
                  _full((DH, 64)), _full((1, 64)), _full((1, 64)),
                  _full((1, 64)),
                  _full((64, 32)), _full((1, 32)), _full((1, 32)),
                  _full((1, 32)),
                  _full((32, 8)), _full((1, 8))],
        out_specs=[pl.BlockSpec((R, 8), lambda i: (i, 0))],
        out_shape=[jax.ShapeDtypeStruct((NROWS, 8), jnp.float32)],
    )(h3, w1, b1, g1, be1, w2, b2, g2, be2, w3, b3)[0]


def _tc_edge_mlp(eea, eeb, edge_p, army_p):
    (we1a, we1b, be1, ge1, bee1), (we2, be2, ge2, bee2), (we3, be3) = edge_p
    (wa1a, wa1b, ba1, ga1, baa1), (wa2, ba2, ga2, baa2), (wa3, ba3) = army_p

    def body(ea, eb,
             e1a, e1b, e1bias, e1g, e1be, e2w, e2b, e2g, e2be, e3w, e3b,
             a1a, a1b, a1bias, a1g, a1be, a2w, a2b, a2g, a2be, a3w, a3b,
             out_ref):
        bf = jnp.bfloat16
        a = jnp.clip(ea[:, :DH], -10.0, 10.0).astype(bf)
        b = jnp.clip(eb[:, :DH], -10.0, 10.0).astype(bf)
        t = _dot(a, e1a[...]) + _dot(b, e1b[...]) + e1bias[...]
        t = jnp.maximum(_ln(t, e1g[...], e1be[...]), 0.0)
        t = jnp.maximum(_ln(_dot(t.astype(bf), e2w[...]) + e2b[...],
                            e2g[...], e2be[...]), 0.0)
        att = jnp.clip(_dot(t.astype(bf), e3w[...]) + e3b[...], -20.0, 20.0)
        u = _dot(a, a1a[...]) + _dot(b, a1b[...]) + a1bias[...]
        u = jnp.maximum(_ln(u, a1g[...], a1be[...]), 0.0)
        u = jnp.maximum(_ln(_dot(u.astype(bf), a2w[...]) + a2b[...],
                            a2g[...], a2be[...]), 0.0)
        army = jnp.clip(_dot(u.astype(bf), a3w[...]) + a3b[...], -20.0,
                        20.0)
        out_ref[...] = jnp.concatenate([att, army], axis=1)

    return pl.pallas_call(
        body,
        grid=(GEB,),
        in_specs=[pl.BlockSpec((R, DW), lambda i: (i, 0)),
                  pl.BlockSpec((R, DW), lambda i: (i, 0)),
                  _full((DH, 64)), _full((DH, 64)), _full((1, 64)),
                  _full((1, 64)), _full((1, 64)),
                  _full((64, 32)), _full((1, 32)), _full((1, 32)),
                  _full((1, 32)),
                  _full((32, 8)), _full((1, 8)),
                  _full((DH, 128)), _full((DH, 128)), _full((1, 128)),
                  _full((1, 128)), _full((1, 128)),
                  _full((128, 64)), _full((1, 64)), _full((1, 64)),
                  _full((1, 64)),
                  _full((64, 8)), _full((1, 8))],
        out_specs=[pl.BlockSpec((R, 16), lambda i: (i, 0))],
        out_shape=[jax.ShapeDtypeStruct((E2B, 16), jnp.float32)],
    )(eea, eeb,
      we1a, we1b, be1, ge1, bee1, we2, be2, ge2, bee2, we3, be3,
      wa1a, wa1b, ba1, ga1, baa1, wa2, ba2, ga2, baa2, wa3, ba3)[0]


def _row(v):
    return v.reshape(1, -1)


def _pad_cols(w, b, cols):
    wp = jnp.zeros((w.shape[0], cols), w.dtype).at[:, :w.shape[1]].set(w)
    bp = jnp.zeros((cols,), b.dtype).at[:b.shape[0]].set(b)
    return wp, _row(bp)


def _head3(p, cols):
    l1, l2, l3 = p
    w3, b3 = _pad_cols(l3["W"], l3["b"], cols)
    return ((l1["W"], _row(l1["b"]), _row(l1["g"]), _row(l1["be"])),
            (l2["W"], _row(l2["b"]), _row(l2["g"]), _row(l2["be"])),
            (w3, b3))


def _head3_split(p, cols):
    l1, l2, l3 = p
    bf = jnp.bfloat16
    w1 = l1["W"].astype(bf)
    w3, b3 = _pad_cols(l3["W"], l3["b"], cols)
    return ((w1[:DH], w1[DH:], _row(l1["b"]), _row(l1["g"]), _row(l1["be"])),
            (l2["W"].astype(bf), _row(l2["b"]), _row(l2["g"]),
             _row(l2["be"])),
            (w3.astype(bf), b3))


def kernel(x, params, action_edges, edge_index):
    n = x.shape[0]
    ne = action_edges.shape[0]
    ei = edge_index.astype(jnp.int32)
    loop = jnp.arange(n, dtype=jnp.int32)
    npad1 = E1 - (ei.shape[1] + n)
    spread = N + jax.lax.rem(jnp.arange(npad1, dtype=jnp.int32),
                             jnp.int32(NROWS - N))
    src1 = jnp.concatenate([ei[0], loop, spread])
    dst1 = jnp.concatenate([ei[1], loop, spread])
    dst3 = dst1.reshape(NW, C1, K)
    ae = action_edges.astype(jnp.int32)
    npad2 = E2 - ne
    spread2 = jax.lax.rem(jnp.arange(npad2, dtype=jnp.int32), jnp.int32(n))
    sidx = jnp.concatenate([jnp.clip(ae[:, 0], 0, n - 1), spread2])
    tidx = jnp.concatenate([jnp.clip(ae[:, 1], 0, n - 1), spread2])
    x_pad = jnp.zeros((NROWS, x.shape[1]), x.dtype).at[:n].set(x)

    gcn = params["gcn"]
    degs = _sc_degree(dst3)
    xw0, id0 = _tc_pre(x_pad, gcn[0]["W"], gcn[0]["P"])
    hw0 = _tc_scale(xw0, degs)
    agg0 = _sc_agg(hw0, src1, dst1)
    h1, hw1 = _tc_post(agg0, degs, id0, _row(gcn[0]["b"]), _row(gcn[0]["g"]),
                       _row(gcn[0]["be"]), gcn[1]["W"], hw0)
    agg1 = _sc_agg(hw1, src1, dst1)
    h2, hw2 = _tc_post(agg1, degs, h1, _row(gcn[1]["b"]), _row(gcn[1]["g"]),
                       _row(gcn[1]["be"]), gcn[2]["W"], hw1)
    agg2 = _sc_agg(hw2, src1, dst1)
    h3 = _tc_post_h3(agg2, degs, h2, _row(gcn[2]["b"]), _row(gcn[2]["g"]),
                     _row(gcn[2]["be"]), hw2)
    edge_p = _head3_split(params["edge"], 8)
    army_p = _head3_split(params["army"], 8)
    ees = [_sc_edge_gather(h3, sidx[k * E2B:(k + 1) * E2B],
                           tidx[k * E2B:(k + 1) * E2B]) for k in range(NB)]
    place8 = _tc_place_head(h3, _head3(params["place"], 8))
    outs = [_tc_edge_mlp(ea, eb, edge_p, army_p) for ea, eb in ees]
    out16 = jnp.concatenate(outs, axis=0)

    placement_logits = place8[:n, 0]
    attack_logits = out16[:ne, 0]
    army_logits = out16[:ne, 8:12]
    return (placement_logits, attack_logits, army_logits)
